# Initial kernel scaffold; baseline (speedup 1.0000x reference)
import jax
import jax.numpy as jnp
from jax.experimental import pallas as pl


def kernel(ip1_idx, W_ip1, ip2_idx, W_ip2, ip3_idx, W_ip3, regionid_idx, W_regionid, cityid_idx, W_cityid, adexchange_idx, W_adexchange, url_idx, W_url, aurl_idx, W_aurl, adslotw_idx, W_adslotw, adsloth_idx, W_adsloth, adslotv_idx, W_adslotv, adslotfp_idx, W_adslotfp, creativeid_idx, W_creativeid, bidprice_idx, W_bidprice, payprice_idx, W_payprice, userids_idx, W_userids):
    raise NotImplementedError("write your pallas kernel here")


# speedup vs baseline: 7.8654x; 7.8654x over previous
import functools

import jax
import jax.numpy as jnp
from jax.experimental import pallas as pl
from jax.experimental.pallas import tpu as pltpu

_FIELDS = [
    ("ip1", 256, 8), ("ip2", 256, 8), ("ip3", 256, 8), ("regionid", 35, 6),
    ("cityid", 370, 9), ("adexchange", 9, 4), ("url", 2, 1), ("aurl", 2, 1),
    ("adslotw", 21, 5), ("adsloth", 14, 4), ("adslotv", 7, 3), ("adslotfp", 275, 9),
    ("creativeid", 57, 6), ("bidprice", 2, 1), ("payprice", 295, 9), ("userids", 69, 7),
]
_NF = len(_FIELDS)
_VOCAB = [v for (_, v, _) in _FIELDS]
_D = [8 * m for (_, _, m) in _FIELDS]
_TOTAL_D = sum(_D)

_B = 16384
_BB = 512
_GRID = _B // _BB


def _body(idx_ref, *refs):
    w_hi = refs[:_NF]
    w_lo = refs[_NF:2 * _NF]
    out_ref = refs[2 * _NF]
    outs = []
    for f in range(_NF):
        idx = idx_ref[f, :]
        iota = jax.lax.broadcasted_iota(jnp.int32, (_BB, _VOCAB[f]), 1)
        oh = (iota == idx[:, None]).astype(jnp.bfloat16)
        hi = jax.lax.dot_general(
            oh, w_hi[f][...], (((1,), (0,)), ((), ())),
            preferred_element_type=jnp.float32)
        lo = jax.lax.dot_general(
            oh, w_lo[f][...], (((1,), (0,)), ((), ())),
            preferred_element_type=jnp.float32)
        outs.append(hi + lo)
    out_ref[...] = jnp.concatenate(outs, axis=1)


@functools.partial(jax.jit, static_argnums=())
def _onehot_embed(idx_all, *tables):
    w_hi = [t.astype(jnp.bfloat16) for t in tables]
    w_lo = [(t - h.astype(jnp.float32)).astype(jnp.bfloat16)
            for t, h in zip(tables, w_hi)]
    grid_spec = pl.GridSpec(
        grid=(_GRID,),
        in_specs=[pl.BlockSpec((_NF, _BB), lambda i: (0, i))]
        + [pl.BlockSpec((v, d), lambda i: (0, 0)) for (v, d) in zip(_VOCAB, _D)]
        + [pl.BlockSpec((v, d), lambda i: (0, 0)) for (v, d) in zip(_VOCAB, _D)],
        out_specs=pl.BlockSpec((_BB, _TOTAL_D), lambda i: (i, 0)),
    )
    return pl.pallas_call(
        _body,
        grid_spec=grid_spec,
        out_shape=jax.ShapeDtypeStruct((_B, _TOTAL_D), jnp.float32),
    )(idx_all, *w_hi, *w_lo)


def kernel(ip1_idx, W_ip1, ip2_idx, W_ip2, ip3_idx, W_ip3, regionid_idx, W_regionid, cityid_idx, W_cityid, adexchange_idx, W_adexchange, url_idx, W_url, aurl_idx, W_aurl, adslotw_idx, W_adslotw, adsloth_idx, W_adsloth, adslotv_idx, W_adslotv, adslotfp_idx, W_adslotfp, creativeid_idx, W_creativeid, bidprice_idx, W_bidprice, payprice_idx, W_payprice, userids_idx, W_userids):
    inp = dict(locals())
    idxs = [inp[name + "_idx"].astype(jnp.int32) for (name, _, _) in _FIELDS]
    tables = [inp["W_" + name] for (name, _, _) in _FIELDS]
    idx_all = jnp.stack(idxs)
    return _onehot_embed(idx_all, *tables)


# speedup vs baseline: 8.1395x; 1.0349x over previous
import functools

import jax
import jax.numpy as jnp
from jax.experimental import pallas as pl
from jax.experimental.pallas import tpu as pltpu

_FIELDS = [
    ("ip1", 256, 8), ("ip2", 256, 8), ("ip3", 256, 8), ("regionid", 35, 6),
    ("cityid", 370, 9), ("adexchange", 9, 4), ("url", 2, 1), ("aurl", 2, 1),
    ("adslotw", 21, 5), ("adsloth", 14, 4), ("adslotv", 7, 3), ("adslotfp", 275, 9),
    ("creativeid", 57, 6), ("bidprice", 2, 1), ("payprice", 295, 9), ("userids", 69, 7),
]
_NF = len(_FIELDS)
_VOCAB = [v for (_, v, _) in _FIELDS]
_D = [8 * m for (_, _, m) in _FIELDS]
_TOTAL_D = sum(_D)

_B = 16384
_BB = 512
_GRID = _B // _BB


def _body(idx_ref, *refs):
    w_hi = refs[:_NF]
    w_lo = refs[_NF:2 * _NF]
    out_ref = refs[2 * _NF]
    outs = []
    for f in range(_NF):
        idx = idx_ref[f, :]
        iota = jax.lax.broadcasted_iota(jnp.int32, (_VOCAB[f], _BB), 0)
        oh_t = (iota == idx[None, :]).astype(jnp.bfloat16)
        hi = jax.lax.dot_general(
            oh_t, w_hi[f][...], (((0,), (0,)), ((), ())),
            preferred_element_type=jnp.float32)
        lo = jax.lax.dot_general(
            oh_t, w_lo[f][...], (((0,), (0,)), ((), ())),
            preferred_element_type=jnp.float32)
        outs.append(hi + lo * (1.0 / 256.0))
    out_ref[...] = jnp.concatenate(outs, axis=1)


@functools.partial(jax.jit, static_argnums=())
def _onehot_embed(idx_all, *tables):
    w_hi = [t.astype(jnp.bfloat16) for t in tables]
    w_lo = [((t - h.astype(jnp.float32)) * 256.0).astype(jnp.bfloat16)
            for t, h in zip(tables, w_hi)]
    grid_spec = pl.GridSpec(
        grid=(_GRID,),
        in_specs=[pl.BlockSpec((_NF, _BB), lambda i: (0, i))]
        + [pl.BlockSpec((v, d), lambda i: (0, 0)) for (v, d) in zip(_VOCAB, _D)]
        + [pl.BlockSpec((v, d), lambda i: (0, 0)) for (v, d) in zip(_VOCAB, _D)],
        out_specs=pl.BlockSpec((_BB, _TOTAL_D), lambda i: (i, 0)),
    )
    return pl.pallas_call(
        _body,
        grid_spec=grid_spec,
        out_shape=jax.ShapeDtypeStruct((_B, _TOTAL_D), jnp.float32),
    )(idx_all, *w_hi, *w_lo)


def kernel(ip1_idx, W_ip1, ip2_idx, W_ip2, ip3_idx, W_ip3, regionid_idx, W_regionid, cityid_idx, W_cityid, adexchange_idx, W_adexchange, url_idx, W_url, aurl_idx, W_aurl, adslotw_idx, W_adslotw, adsloth_idx, W_adsloth, adslotv_idx, W_adslotv, adslotfp_idx, W_adslotfp, creativeid_idx, W_creativeid, bidprice_idx, W_bidprice, payprice_idx, W_payprice, userids_idx, W_userids):
    inp = dict(locals())
    idxs = [inp[name + "_idx"].astype(jnp.int32) for (name, _, _) in _FIELDS]
    tables = [inp["W_" + name] for (name, _, _) in _FIELDS]
    idx_all = jnp.stack(idxs)
    return _onehot_embed(idx_all, *tables)


# speedup vs baseline: 8.8056x; 1.0818x over previous
import functools

import jax
import jax.numpy as jnp
from jax.experimental import pallas as pl
from jax.experimental.pallas import tpu as pltpu

_FIELDS = [
    ("ip1", 256, 8), ("ip2", 256, 8), ("ip3", 256, 8), ("regionid", 35, 6),
    ("cityid", 370, 9), ("adexchange", 9, 4), ("url", 2, 1), ("aurl", 2, 1),
    ("adslotw", 21, 5), ("adsloth", 14, 4), ("adslotv", 7, 3), ("adslotfp", 275, 9),
    ("creativeid", 57, 6), ("bidprice", 2, 1), ("payprice", 295, 9), ("userids", 69, 7),
]
_NF = len(_FIELDS)
_VOCAB = [v for (_, v, _) in _FIELDS]
_D = [8 * m for (_, _, m) in _FIELDS]
_TOTAL_D = sum(_D)

_B = 16384
_BB = 512
_GRID = _B // _BB


def _body(idx_ref, *refs):
    w_hi = refs[:_NF]
    out_ref = refs[_NF]
    outs = []
    for f in range(_NF):
        idx = idx_ref[f, :]
        iota = jax.lax.broadcasted_iota(jnp.int32, (_VOCAB[f], _BB), 0)
        oh_t = (iota == idx[None, :]).astype(jnp.bfloat16)
        outs.append(jax.lax.dot_general(
            oh_t, w_hi[f][...], (((0,), (0,)), ((), ())),
            preferred_element_type=jnp.float32))
    out_ref[...] = jnp.concatenate(outs, axis=1)


@functools.partial(jax.jit, static_argnums=())
def _onehot_embed(idx_all, *tables):
    w_hi = [t.astype(jnp.bfloat16) for t in tables]
    grid_spec = pl.GridSpec(
        grid=(_GRID,),
        in_specs=[pl.BlockSpec((_NF, _BB), lambda i: (0, i))]
        + [pl.BlockSpec((v, d), lambda i: (0, 0)) for (v, d) in zip(_VOCAB, _D)],
        out_specs=pl.BlockSpec((_BB, _TOTAL_D), lambda i: (i, 0)),
    )
    return pl.pallas_call(
        _body,
        grid_spec=grid_spec,
        out_shape=jax.ShapeDtypeStruct((_B, _TOTAL_D), jnp.float32),
    )(idx_all, *w_hi)


def kernel(ip1_idx, W_ip1, ip2_idx, W_ip2, ip3_idx, W_ip3, regionid_idx, W_regionid, cityid_idx, W_cityid, adexchange_idx, W_adexchange, url_idx, W_url, aurl_idx, W_aurl, adslotw_idx, W_adslotw, adsloth_idx, W_adsloth, adslotv_idx, W_adslotv, adslotfp_idx, W_adslotfp, creativeid_idx, W_creativeid, bidprice_idx, W_bidprice, payprice_idx, W_payprice, userids_idx, W_userids):
    inp = dict(locals())
    idxs = [inp[name + "_idx"].astype(jnp.int32) for (name, _, _) in _FIELDS]
    tables = [inp["W_" + name] for (name, _, _) in _FIELDS]
    idx_all = jnp.stack(idxs)
    return _onehot_embed(idx_all, *tables)


# speedup vs baseline: 9.9766x; 1.1330x over previous
import functools

import jax
import jax.numpy as jnp
from jax.experimental import pallas as pl

_FIELDS = [
    ("ip1", 256, 8), ("ip2", 256, 8), ("ip3", 256, 8), ("regionid", 35, 6),
    ("cityid", 370, 9), ("adexchange", 9, 4), ("url", 2, 1), ("aurl", 2, 1),
    ("adslotw", 21, 5), ("adsloth", 14, 4), ("adslotv", 7, 3), ("adslotfp", 275, 9),
    ("creativeid", 57, 6), ("bidprice", 2, 1), ("payprice", 295, 9), ("userids", 69, 7),
]
_NF = len(_FIELDS)
_VOCAB = [v for (_, v, _) in _FIELDS]
_V16 = [-(-v // 16) * 16 for v in _VOCAB]
_D = [8 * m for (_, _, m) in _FIELDS]
_OFF = [0]
for _d in _D[:-1]:
    _OFF.append(_OFF[-1] + _d)
_TOTAL_D = _OFF[-1] + _D[-1]

_WINDOWS = []
for _w0 in range(0, _TOTAL_D, 128):
    _w1 = min(_w0 + 128, _TOTAL_D)
    _pieces, _rows = [], 0
    for _f in range(_NF):
        _s, _e = max(_OFF[_f], _w0), min(_OFF[_f] + _D[_f], _w1)
        if _s < _e:
            _pieces.append((_f, _s - _OFF[_f], _e - _OFF[_f], _s - _w0, _rows))
            _rows += _V16[_f]
    _WINDOWS.append((_w1 - _w0, _rows, _pieces))

_B = 16384
_BB = 512
_GRID = _B // _BB


def _body(idx_ref, *refs):
    w_refs = refs[:len(_WINDOWS)]
    out_ref = refs[len(_WINDOWS)]
    col = 0
    for wi, (width, rows, pieces) in enumerate(_WINDOWS):
        ohs = []
        for (f, _cs, _ce, _cw, _rb) in pieces:
            idx = idx_ref[f, :]
            iota = jax.lax.broadcasted_iota(jnp.int32, (_V16[f], _BB), 0)
            ohs.append((iota == idx[None, :]).astype(jnp.bfloat16))
        oh = ohs[0] if len(ohs) == 1 else jnp.concatenate(ohs, axis=0)
        res = jax.lax.dot_general(
            oh, w_refs[wi][...], (((0,), (0,)), ((), ())),
            preferred_element_type=jnp.float32)
        out_ref[:, col:col + width] = res
        col += width


@jax.jit
def _onehot_embed(idx_all, *wbins):
    grid_spec = pl.GridSpec(
        grid=(_GRID,),
        in_specs=[pl.BlockSpec((_NF, _BB), lambda i: (0, i))]
        + [pl.BlockSpec((r, w), lambda i: (0, 0)) for (w, r, _) in _WINDOWS],
        out_specs=pl.BlockSpec((_BB, _TOTAL_D), lambda i: (i, 0)),
    )
    return pl.pallas_call(
        _body,
        grid_spec=grid_spec,
        out_shape=jax.ShapeDtypeStruct((_B, _TOTAL_D), jnp.float32),
    )(idx_all, *wbins)


def kernel(ip1_idx, W_ip1, ip2_idx, W_ip2, ip3_idx, W_ip3, regionid_idx, W_regionid, cityid_idx, W_cityid, adexchange_idx, W_adexchange, url_idx, W_url, aurl_idx, W_aurl, adslotw_idx, W_adslotw, adsloth_idx, W_adsloth, adslotv_idx, W_adslotv, adslotfp_idx, W_adslotfp, creativeid_idx, W_creativeid, bidprice_idx, W_bidprice, payprice_idx, W_payprice, userids_idx, W_userids):
    inp = dict(locals())
    idxs = [inp[name + "_idx"].astype(jnp.int32) for (name, _, _) in _FIELDS]
    tables = [inp["W_" + name] for (name, _, _) in _FIELDS]
    idx_all = jnp.stack(idxs)
    wbins = []
    for (width, rows, pieces) in _WINDOWS:
        m = jnp.zeros((rows, width), jnp.float32)
        for (f, cs, ce, cw, rb) in pieces:
            m = m.at[rb:rb + _VOCAB[f], cw:cw + (ce - cs)].set(tables[f][:, cs:ce])
        wbins.append(m.astype(jnp.bfloat16))
    return _onehot_embed(idx_all, *wbins)


# speedup vs baseline: 11.7312x; 1.1759x over previous
import jax
import jax.numpy as jnp
from jax.experimental import pallas as pl
from jax.experimental.pallas import tpu as pltpu

_FIELDS = [
    ("ip1", 256, 8), ("ip2", 256, 8), ("ip3", 256, 8), ("regionid", 35, 6),
    ("cityid", 370, 9), ("adexchange", 9, 4), ("url", 2, 1), ("aurl", 2, 1),
    ("adslotw", 21, 5), ("adsloth", 14, 4), ("adslotv", 7, 3), ("adslotfp", 275, 9),
    ("creativeid", 57, 6), ("bidprice", 2, 1), ("payprice", 295, 9), ("userids", 69, 7),
]
_NF = len(_FIELDS)
_VOCAB = [v for (_, v, _) in _FIELDS]
_V16 = [-(-v // 16) * 16 for v in _VOCAB]
_D = [8 * m for (_, _, m) in _FIELDS]
_OFF = [0]
for _d in _D[:-1]:
    _OFF.append(_OFF[-1] + _d)
_TOTAL_D = _OFF[-1] + _D[-1]

_WINDOWS = []
for _w0 in range(0, _TOTAL_D, 128):
    _w1 = min(_w0 + 128, _TOTAL_D)
    _pieces, _rows = [], 0
    for _f in range(_NF):
        _s, _e = max(_OFF[_f], _w0), min(_OFF[_f] + _D[_f], _w1)
        if _s < _e:
            _pieces.append((_f, _s - _OFF[_f], _e - _OFF[_f], _s - _w0, _rows))
            _rows += _V16[_f]
    _WINDOWS.append((_w1 - _w0, _rows, _pieces))
_NW = len(_WINDOWS)

_B = 16384
_BB = 512
_GRID = _B // _BB


def _body(idx_ref, *refs):
    t_refs = refs[:_NF]
    out_ref = refs[_NF]
    bins = refs[_NF + 1:]

    @pl.when(pl.program_id(0) == 0)
    def _():
        for wi, (width, rows, pieces) in enumerate(_WINDOWS):
            bins[wi][...] = jnp.zeros((rows, width), jnp.bfloat16)
            for (f, cs, ce, cw, rb) in pieces:
                bins[wi][rb:rb + _VOCAB[f], cw:cw + (ce - cs)] = (
                    t_refs[f][:, cs:ce].astype(jnp.bfloat16))

    col = 0
    for wi, (width, rows, pieces) in enumerate(_WINDOWS):
        ohs = []
        for (f, _cs, _ce, _cw, _rb) in pieces:
            idx = idx_ref[f, :]
            iota = jax.lax.broadcasted_iota(jnp.int32, (_V16[f], _BB), 0)
            ohs.append((iota == idx[None, :]).astype(jnp.bfloat16))
        oh = ohs[0] if len(ohs) == 1 else jnp.concatenate(ohs, axis=0)
        res = jax.lax.dot_general(
            oh, bins[wi][...], (((0,), (0,)), ((), ())),
            preferred_element_type=jnp.float32)
        out_ref[:, col:col + width] = res
        col += width


@jax.jit
def _onehot_embed(idx_all, *tables):
    return pl.pallas_call(
        _body,
        grid=(_GRID,),
        in_specs=[pl.BlockSpec((_NF, _BB), lambda i: (0, i))]
        + [pl.BlockSpec((v, d), lambda i: (0, 0)) for (v, d) in zip(_VOCAB, _D)],
        out_specs=pl.BlockSpec((_BB, _TOTAL_D), lambda i: (i, 0)),
        out_shape=jax.ShapeDtypeStruct((_B, _TOTAL_D), jnp.float32),
        scratch_shapes=[pltpu.VMEM((r, w), jnp.bfloat16) for (w, r, _) in _WINDOWS],
    )(idx_all, *tables)


def kernel(ip1_idx, W_ip1, ip2_idx, W_ip2, ip3_idx, W_ip3, regionid_idx, W_regionid, cityid_idx, W_cityid, adexchange_idx, W_adexchange, url_idx, W_url, aurl_idx, W_aurl, adslotw_idx, W_adslotw, adsloth_idx, W_adsloth, adslotv_idx, W_adslotv, adslotfp_idx, W_adslotfp, creativeid_idx, W_creativeid, bidprice_idx, W_bidprice, payprice_idx, W_payprice, userids_idx, W_userids):
    inp = dict(locals())
    idxs = [inp[name + "_idx"].astype(jnp.int32) for (name, _, _) in _FIELDS]
    tables = [inp["W_" + name] for (name, _, _) in _FIELDS]
    idx_all = jnp.stack(idxs)
    return _onehot_embed(idx_all, *tables)


# speedup vs baseline: 12.2858x; 1.0473x over previous
import jax
import jax.numpy as jnp
from jax.experimental import pallas as pl
from jax.experimental.pallas import tpu as pltpu

_FIELDS = [
    ("ip1", 256, 8), ("ip2", 256, 8), ("ip3", 256, 8), ("regionid", 35, 6),
    ("cityid", 370, 9), ("adexchange", 9, 4), ("url", 2, 1), ("aurl", 2, 1),
    ("adslotw", 21, 5), ("adsloth", 14, 4), ("adslotv", 7, 3), ("adslotfp", 275, 9),
    ("creativeid", 57, 6), ("bidprice", 2, 1), ("payprice", 295, 9), ("userids", 69, 7),
]
_NF = len(_FIELDS)
_VOCAB = [v for (_, v, _) in _FIELDS]
_V16 = [-(-v // 16) * 16 for v in _VOCAB]
_D = [8 * m for (_, _, m) in _FIELDS]
_OFF = [0]
for _d in _D[:-1]:
    _OFF.append(_OFF[-1] + _d)
_TOTAL_D = _OFF[-1] + _D[-1]

_WINDOWS = []
for _w0 in range(0, _TOTAL_D, 128):
    _w1 = min(_w0 + 128, _TOTAL_D)
    _pieces, _rows = [], 0
    for _f in range(_NF):
        _s, _e = max(_OFF[_f], _w0), min(_OFF[_f] + _D[_f], _w1)
        if _s < _e:
            _pieces.append((_f, _s - _OFF[_f], _e - _OFF[_f], _s - _w0, _rows))
            _rows += _V16[_f]
    _WINDOWS.append((_w1 - _w0, _rows, _pieces))
_NW = len(_WINDOWS)

_B = 16384
_BB = 1024
_GRID = _B // _BB


def _body(idx_ref, *refs):
    t_refs = refs[:_NF]
    out_ref = refs[_NF]
    bins = refs[_NF + 1:]

    @pl.when(pl.program_id(0) == 0)
    def _():
        for wi, (width, rows, pieces) in enumerate(_WINDOWS):
            bins[wi][...] = jnp.zeros((rows, width), jnp.bfloat16)
            for (f, cs, ce, cw, rb) in pieces:
                bins[wi][rb:rb + _VOCAB[f], cw:cw + (ce - cs)] = (
                    t_refs[f][:, cs:ce].astype(jnp.bfloat16))

    oh_cache = {}

    def field_oh(f):
        if f not in oh_cache:
            idx = idx_ref[f, :]
            iota = jax.lax.broadcasted_iota(jnp.int32, (_V16[f], _BB), 0)
            oh_cache[f] = (iota == idx[None, :]).astype(jnp.bfloat16)
        return oh_cache[f]

    col = 0
    for wi, (width, rows, pieces) in enumerate(_WINDOWS):
        ohs = [field_oh(f) for (f, _cs, _ce, _cw, _rb) in pieces]
        oh = ohs[0] if len(ohs) == 1 else jnp.concatenate(ohs, axis=0)
        res = jax.lax.dot_general(
            oh, bins[wi][...], (((0,), (0,)), ((), ())),
            preferred_element_type=jnp.float32)
        out_ref[:, col:col + width] = res
        col += width


@jax.jit
def _onehot_embed(idx_all, *tables):
    return pl.pallas_call(
        _body,
        grid=(_GRID,),
        in_specs=[pl.BlockSpec((_NF, _BB), lambda i: (0, i))]
        + [pl.BlockSpec((v, d), lambda i: (0, 0)) for (v, d) in zip(_VOCAB, _D)],
        out_specs=pl.BlockSpec((_BB, _TOTAL_D), lambda i: (i, 0)),
        out_shape=jax.ShapeDtypeStruct((_B, _TOTAL_D), jnp.float32),
        scratch_shapes=[pltpu.VMEM((r, w), jnp.bfloat16) for (w, r, _) in _WINDOWS],
    )(idx_all, *tables)


def kernel(ip1_idx, W_ip1, ip2_idx, W_ip2, ip3_idx, W_ip3, regionid_idx, W_regionid, cityid_idx, W_cityid, adexchange_idx, W_adexchange, url_idx, W_url, aurl_idx, W_aurl, adslotw_idx, W_adslotw, adsloth_idx, W_adsloth, adslotv_idx, W_adslotv, adslotfp_idx, W_adslotfp, creativeid_idx, W_creativeid, bidprice_idx, W_bidprice, payprice_idx, W_payprice, userids_idx, W_userids):
    inp = dict(locals())
    idxs = [inp[name + "_idx"].astype(jnp.int32) for (name, _, _) in _FIELDS]
    tables = [inp["W_" + name] for (name, _, _) in _FIELDS]
    idx_all = jnp.stack(idxs)
    return _onehot_embed(idx_all, *tables)


# speedup vs baseline: 13.3501x; 1.0866x over previous
import jax
import jax.numpy as jnp
from jax.experimental import pallas as pl
from jax.experimental.pallas import tpu as pltpu

_FIELDS = [
    ("ip1", 256, 8), ("ip2", 256, 8), ("ip3", 256, 8), ("regionid", 35, 6),
    ("cityid", 370, 9), ("adexchange", 9, 4), ("url", 2, 1), ("aurl", 2, 1),
    ("adslotw", 21, 5), ("adsloth", 14, 4), ("adslotv", 7, 3), ("adslotfp", 275, 9),
    ("creativeid", 57, 6), ("bidprice", 2, 1), ("payprice", 295, 9), ("userids", 69, 7),
]
_NF = len(_FIELDS)
_VOCAB = [v for (_, v, _) in _FIELDS]
_V16 = [-(-v // 16) * 16 for v in _VOCAB]
_D = [8 * m for (_, _, m) in _FIELDS]
_OFF = [0]
for _d in _D[:-1]:
    _OFF.append(_OFF[-1] + _d)
_TOTAL_D = _OFF[-1] + _D[-1]

_WINDOWS = []
for _w0 in range(0, _TOTAL_D, 128):
    _w1 = min(_w0 + 128, _TOTAL_D)
    _pieces, _rows = [], 0
    for _f in range(_NF):
        _s, _e = max(_OFF[_f], _w0), min(_OFF[_f] + _D[_f], _w1)
        if _s < _e:
            _pieces.append((_f, _s - _OFF[_f], _e - _OFF[_f], _s - _w0, _rows))
            _rows += _V16[_f]
    _WINDOWS.append((_w1 - _w0, _rows, _pieces))
_NW = len(_WINDOWS)

_B = 16384
_BB = 1024
_GRID = _B // _BB


def _body(idx_ref, *refs):
    t_refs = refs[:_NF]
    out_ref = refs[_NF]
    bins = refs[_NF + 1:]

    @pl.when(pl.program_id(0) == 0)
    def _():
        for wi, (width, rows, pieces) in enumerate(_WINDOWS):
            bins[wi][...] = jnp.zeros((rows, width), jnp.bfloat16)
            for (f, cs, ce, cw, rb) in pieces:
                bins[wi][rb:rb + _VOCAB[f], cw:cw + (ce - cs)] = (
                    t_refs[f][:, cs:ce].astype(jnp.bfloat16))

    oh_cache = {}

    def field_oh(f):
        if f not in oh_cache:
            idx = idx_ref[f, :]
            iota = jax.lax.broadcasted_iota(jnp.int32, (_V16[f], _BB), 0)
            oh_cache[f] = (iota == idx[None, :]).astype(jnp.bfloat16)
        return oh_cache[f]

    col = 0
    for wi, (width, rows, pieces) in enumerate(_WINDOWS):
        ohs = [field_oh(f) for (f, _cs, _ce, _cw, _rb) in pieces]
        oh = ohs[0] if len(ohs) == 1 else jnp.concatenate(ohs, axis=0)
        res_t = jax.lax.dot_general(
            bins[wi][...], oh, (((0,), (0,)), ((), ())),
            preferred_element_type=jnp.float32)
        out_ref[:, col:col + width] = res_t.T
        col += width


@jax.jit
def _onehot_embed(idx_all, *tables):
    return pl.pallas_call(
        _body,
        grid=(_GRID,),
        in_specs=[pl.BlockSpec((_NF, _BB), lambda i: (0, i))]
        + [pl.BlockSpec((v, d), lambda i: (0, 0)) for (v, d) in zip(_VOCAB, _D)],
        out_specs=pl.BlockSpec((_BB, _TOTAL_D), lambda i: (i, 0)),
        out_shape=jax.ShapeDtypeStruct((_B, _TOTAL_D), jnp.float32),
        scratch_shapes=[pltpu.VMEM((r, w), jnp.bfloat16) for (w, r, _) in _WINDOWS],
    )(idx_all, *tables)


def kernel(ip1_idx, W_ip1, ip2_idx, W_ip2, ip3_idx, W_ip3, regionid_idx, W_regionid, cityid_idx, W_cityid, adexchange_idx, W_adexchange, url_idx, W_url, aurl_idx, W_aurl, adslotw_idx, W_adslotw, adsloth_idx, W_adsloth, adslotv_idx, W_adslotv, adslotfp_idx, W_adslotfp, creativeid_idx, W_creativeid, bidprice_idx, W_bidprice, payprice_idx, W_payprice, userids_idx, W_userids):
    inp = dict(locals())
    idxs = [inp[name + "_idx"].astype(jnp.int32) for (name, _, _) in _FIELDS]
    tables = [inp["W_" + name] for (name, _, _) in _FIELDS]
    idx_all = jnp.stack(idxs)
    return _onehot_embed(idx_all, *tables)
